# baseline (device time: 13348 ns/iter reference)
import jax
import jax.numpy as jnp
from jax import lax
from jax.experimental import pallas as pl
from jax.experimental.pallas import tpu as pltpu

N_DEV = 4
CHUNK = 512


def kernel(x):
    m_per, n = x.shape
    m_global = N_DEV * m_per
    n_chunks = m_per // CHUNK

    def body(x_hbm, out_ref, buf_ref, copy_sems, acc_ref, send_sems, recv_sems):
        my = lax.axis_index("i")

        barrier_sem = pltpu.get_barrier_semaphore()
        for off in range(1, N_DEV):
            peer = lax.rem(my + off, N_DEV)
            pl.semaphore_signal(
                barrier_sem,
                inc=1,
                device_id=(peer,),
                device_id_type=pl.DeviceIdType.MESH,
            )
        pl.semaphore_wait(barrier_sem, N_DEV - 1)

        def start_copy(h):
            cp = pltpu.make_async_copy(
                x_hbm.at[pl.ds(h * CHUNK, CHUNK), :],
                buf_ref.at[h % 2],
                copy_sems.at[h % 2],
            )
            cp.start()
            return cp

        copies = [start_copy(0)]
        total = jnp.zeros((1, n), jnp.float32)
        for h in range(n_chunks):
            if h + 1 < n_chunks:
                copies.append(start_copy(h + 1))
            copies[h].wait()
            total = total + jnp.sum(buf_ref[h % 2], axis=0, keepdims=True)
        acc_ref[0, :, :] = total

        rdmas = []
        for off in range(1, N_DEV):
            peer = lax.rem(my + off, N_DEV)
            slot = N_DEV - off
            rdma = pltpu.make_async_remote_copy(
                src_ref=acc_ref.at[0],
                dst_ref=acc_ref.at[slot],
                send_sem=send_sems.at[off - 1],
                recv_sem=recv_sems.at[slot],
                device_id=(peer,),
                device_id_type=pl.DeviceIdType.MESH,
            )
            rdma.start()
            rdmas.append(rdma)

        for rdma in rdmas:
            rdma.wait_recv()

        total = acc_ref[0] + acc_ref[1] + acc_ref[2] + acc_ref[3]
        out_ref[:, :] = total * (1.0 / m_global)

        for rdma in rdmas:
            rdma.wait_send()

    return pl.pallas_call(
        body,
        out_shape=jax.ShapeDtypeStruct((1, n), jnp.float32),
        in_specs=[pl.BlockSpec(memory_space=pl.ANY)],
        out_specs=pl.BlockSpec(memory_space=pltpu.VMEM),
        scratch_shapes=[
            pltpu.VMEM((2, CHUNK, n), jnp.float32),
            pltpu.SemaphoreType.DMA((2,)),
            pltpu.VMEM((N_DEV, 1, n), jnp.float32),
            pltpu.SemaphoreType.DMA((N_DEV - 1,)),
            pltpu.SemaphoreType.DMA((N_DEV,)),
        ],
        compiler_params=pltpu.CompilerParams(collective_id=0),
    )(x)
